# baseline (speedup 1.0000x reference)
import functools

import jax
import jax.numpy as jnp
from jax import lax
from jax.experimental import pallas as pl
from jax.experimental.pallas import tpu as pltpu
from jax.experimental.pallas import tpu_sc as plsc

_NW = 32
_CHUNK = 128
_NBUF = 4


def _prep_body(cat_ref, Wvis_ref, Wself_ref, Wnbr_ref, bvis_ref, bconv_ref,
               AB_ref, bias_ref, TT_ref, emb, hid):
    Wsv = Wself_ref[emb:, :]
    Wnv = Wnbr_ref[emb:, :]
    Wvis = Wvis_ref[...]
    AB_ref[:, :hid] = jnp.dot(Wvis, Wsv, preferred_element_type=jnp.float32)
    AB_ref[:, hid:] = jnp.dot(Wvis, Wnv, preferred_element_type=jnp.float32)
    bias_ref[...] = (jnp.dot(bvis_ref[...], Wsv + Wnv,
                             preferred_element_type=jnp.float32)
                     + bconv_ref[...])
    cat = cat_ref[...]
    TTs = jnp.dot(cat, Wself_ref[:emb, :], preferred_element_type=jnp.float32)
    TTn = jnp.dot(cat, Wnbr_ref[:emb, :], preferred_element_type=jnp.float32)
    tsu = lax.bitcast_convert_type(
        TTs.astype(jnp.bfloat16).astype(jnp.float32), jnp.uint32) >> 16
    tnu = lax.bitcast_convert_type(
        TTn.astype(jnp.bfloat16).astype(jnp.float32),
        jnp.uint32) & jnp.uint32(0xFFFF0000)
    TT_ref[...] = lax.bitcast_convert_type(tsu | tnu, jnp.int32)


def _sc_gather_body(tbl_ref, idx1_ref, idx2_ref, e1_ref, e2_ref,
                    idx_v, rows_v, s0, s1, s2, s3, b_per_w):
    sems = [s0, s1, s2, s3]
    wid = lax.axis_index("s") * 2 + lax.axis_index("c")
    base = wid * b_per_w
    jobs = []
    for t in range(2):
        idx_hbm = idx1_ref if t == 0 else idx2_ref
        out_hbm = e1_ref if t == 0 else e2_ref
        for c in range(b_per_w // _CHUNK):
            jobs.append((idx_hbm, out_hbm, c * _CHUNK))

    def fire(j, b):
        idx_hbm, _, off = jobs[j]
        pltpu.sync_copy(idx_hbm.at[pl.ds(base + off, _CHUNK)], idx_v.at[b])
        return pltpu.async_copy(tbl_ref.at[idx_v.at[b]], rows_v.at[b],
                                sems[b])

    handles = [None] * _NBUF
    for b in range(min(_NBUF, len(jobs))):
        handles[b] = fire(b, b)
    for j in range(len(jobs)):
        b = j % _NBUF
        handles[b].wait()
        _, out_hbm, off = jobs[j]
        pltpu.sync_copy(rows_v.at[b], out_hbm.at[pl.ds(base + off, _CHUNK)])
        nj = j + _NBUF
        if nj < len(jobs):
            handles[b] = fire(nj, b)


def _unpack(g):
    a = lax.bitcast_convert_type(g << 16, jnp.float32)
    b = lax.bitcast_convert_type(g & jnp.int32(-65536), jnp.float32)
    return a, b


def _main_body(v1_ref, v2_ref, g1_ref, g2_ref, AB_ref,
               bias_ref, W1_ref, b1_ref, w2_ref, b2_ref, out_ref, hid):
    A1, B1 = _unpack(g1_ref[...])
    A2, B2 = _unpack(g2_ref[...])
    AB = AB_ref[...]
    P1 = jnp.dot(v1_ref[...], AB, preferred_element_type=jnp.float32)
    P2 = jnp.dot(v2_ref[...], AB, preferred_element_type=jnp.float32)
    b = bias_ref[...]
    u = A1 + B2 + P1[:, :hid] + P2[:, hid:] + b
    w = A2 + B1 + P2[:, :hid] + P1[:, hid:] + b
    pooled = (jnp.maximum(u, 0.0) + jnp.maximum(w, 0.0)) * 0.5
    hid_act = jnp.maximum(
        jnp.dot(pooled, W1_ref[...], preferred_element_type=jnp.float32)
        + b1_ref[...], 0.0)
    score = (jnp.dot(hid_act, w2_ref[...], preferred_element_type=jnp.float32)
             + b2_ref[...])
    out_ref[0] = score.T


def kernel(cat_id1, visual1, cat_id2, visual2, cat_table, W_vis, b_vis,
           W_self, W_nbr, b_conv, W1, b1, w2, b2):
    n, vis = visual1.shape
    ncat, emb = cat_table.shape
    hid = W1.shape[0]
    R = 1000
    nb = n // R

    n_pad = ((n + _NW * _CHUNK - 1) // (_NW * _CHUNK)) * (_NW * _CHUNK)
    b_per_w = n_pad // _NW

    c1 = jnp.pad(cat_id1.astype(jnp.int32), (0, n_pad - n))
    c2 = jnp.pad(cat_id2.astype(jnp.int32), (0, n_pad - n))

    AB, bias, TT = pl.pallas_call(
        lambda *refs: _prep_body(*refs, emb=emb, hid=hid),
        out_shape=[
            jax.ShapeDtypeStruct((vis, 2 * hid), jnp.float32),
            jax.ShapeDtypeStruct((1, hid), jnp.float32),
            jax.ShapeDtypeStruct((ncat, hid), jnp.int32),
        ],
    )(cat_table, W_vis, W_self, W_nbr,
      b_vis.reshape(1, emb), b_conv.reshape(1, hid))

    mesh = plsc.VectorSubcoreMesh(core_axis_name="c", subcore_axis_name="s")
    g1, g2 = pl.kernel(
        functools.partial(_sc_gather_body, b_per_w=b_per_w),
        mesh=mesh,
        out_type=[
            jax.ShapeDtypeStruct((n_pad, hid), jnp.int32),
            jax.ShapeDtypeStruct((n_pad, hid), jnp.int32),
        ],
        scratch_types=[
            pltpu.VMEM((_NBUF, _CHUNK), jnp.int32),
            pltpu.VMEM((_NBUF, _CHUNK, hid), jnp.int32),
            pltpu.SemaphoreType.DMA,
            pltpu.SemaphoreType.DMA,
            pltpu.SemaphoreType.DMA,
            pltpu.SemaphoreType.DMA,
        ],
    )(TT, c1, c2)

    grid = (nb,)
    out = pl.pallas_call(
        lambda *refs: _main_body(*refs, hid=hid),
        grid=grid,
        in_specs=[
            pl.BlockSpec((R, vis), lambda i: (i, 0)),
            pl.BlockSpec((R, vis), lambda i: (i, 0)),
            pl.BlockSpec((R, hid), lambda i: (i, 0)),
            pl.BlockSpec((R, hid), lambda i: (i, 0)),
            pl.BlockSpec((vis, 2 * hid), lambda i: (0, 0)),
            pl.BlockSpec((1, hid), lambda i: (0, 0)),
            pl.BlockSpec((hid, hid), lambda i: (0, 0)),
            pl.BlockSpec((1, hid), lambda i: (0, 0)),
            pl.BlockSpec((hid, 1), lambda i: (0, 0)),
            pl.BlockSpec((1, 1), lambda i: (0, 0)),
        ],
        out_specs=pl.BlockSpec((1, 1, R), lambda i: (i, 0, 0)),
        out_shape=jax.ShapeDtypeStruct((nb, 1, R), jnp.float32),
    )(visual1, visual2, g1, g2, AB, bias, W1,
      b1.reshape(1, hid), w2.reshape(hid, 1), b2.reshape(1, 1))

    return out.reshape(n)


# speedup vs baseline: 1.0057x; 1.0057x over previous
import functools

import jax
import jax.numpy as jnp
from jax import lax
from jax.experimental import pallas as pl
from jax.experimental.pallas import tpu as pltpu
from jax.experimental.pallas import tpu_sc as plsc

_NW = 32
_CHUNK = 128
_NBUF = 4


def _prep_body(cat_ref, Wvis_ref, Wself_ref, Wnbr_ref, bvis_ref, bconv_ref,
               AB_ref, bias_ref, TT_ref, emb, hid):
    Wsv = Wself_ref[emb:, :]
    Wnv = Wnbr_ref[emb:, :]
    Wvis = Wvis_ref[...]
    AB_ref[:, :hid] = jnp.dot(Wvis, Wsv, preferred_element_type=jnp.float32)
    AB_ref[:, hid:] = jnp.dot(Wvis, Wnv, preferred_element_type=jnp.float32)
    bias_ref[...] = (jnp.dot(bvis_ref[...], Wsv + Wnv,
                             preferred_element_type=jnp.float32)
                     + bconv_ref[...])
    cat = cat_ref[...]
    TTs = jnp.dot(cat, Wself_ref[:emb, :], preferred_element_type=jnp.float32)
    TTn = jnp.dot(cat, Wnbr_ref[:emb, :], preferred_element_type=jnp.float32)
    tsu = lax.bitcast_convert_type(
        TTs.astype(jnp.bfloat16).astype(jnp.float32), jnp.uint32) >> 16
    tnu = lax.bitcast_convert_type(
        TTn.astype(jnp.bfloat16).astype(jnp.float32),
        jnp.uint32) & jnp.uint32(0xFFFF0000)
    TT_ref[...] = lax.bitcast_convert_type(tsu | tnu, jnp.int32)


def _sc_gather_body(tbl_ref, idx1_ref, idx2_ref, e1_ref, e2_ref,
                    idxall_v, rows_v, g0, g1, g2, g3, t0, t1, t2, t3,
                    b_per_w):
    gsems = [g0, g1, g2, g3]
    ssems = [t0, t1, t2, t3]
    wid = lax.axis_index("s") * 2 + lax.axis_index("c")
    base = wid * b_per_w
    pltpu.sync_copy(idx1_ref.at[pl.ds(base, b_per_w)], idxall_v.at[0])
    pltpu.sync_copy(idx2_ref.at[pl.ds(base, b_per_w)], idxall_v.at[1])
    jobs = []
    for t in range(2):
        out_hbm = e1_ref if t == 0 else e2_ref
        for c in range(b_per_w // _CHUNK):
            jobs.append((t, out_hbm, c * _CHUNK))

    def fire(j, b):
        t, _, off = jobs[j]
        return pltpu.async_copy(
            tbl_ref.at[idxall_v.at[t, pl.ds(off, _CHUNK)]],
            rows_v.at[b], gsems[b])

    nj_total = len(jobs)
    gh = [None] * _NBUF
    sh = [None] * _NBUF
    for b in range(min(_NBUF, nj_total)):
        gh[b] = fire(b, b)
    for j in range(nj_total):
        b = j % _NBUF
        fj = j + _NBUF - 1
        if j >= 1 and fj < nj_total:
            fb = fj % _NBUF
            sh[fb].wait()
            gh[fb] = fire(fj, fb)
        gh[b].wait()
        _, out_hbm, off = jobs[j]
        sh[b] = pltpu.async_copy(rows_v.at[b],
                                 out_hbm.at[pl.ds(base + off, _CHUNK)],
                                 ssems[b])
    for j in range(max(0, nj_total - (_NBUF - 1)), nj_total):
        sh[j % _NBUF].wait()


def _unpack(g):
    a = lax.bitcast_convert_type(g << 16, jnp.float32)
    b = lax.bitcast_convert_type(g & jnp.int32(-65536), jnp.float32)
    return a, b


def _main_body(v1_ref, v2_ref, g1_ref, g2_ref, AB_ref,
               bias_ref, W1_ref, b1_ref, w2_ref, b2_ref, out_ref, hid):
    A1, B1 = _unpack(g1_ref[...])
    A2, B2 = _unpack(g2_ref[...])
    AB = AB_ref[...]
    P1 = jnp.dot(v1_ref[...], AB, preferred_element_type=jnp.float32)
    P2 = jnp.dot(v2_ref[...], AB, preferred_element_type=jnp.float32)
    b = bias_ref[...]
    u = A1 + B2 + P1[:, :hid] + P2[:, hid:] + b
    w = A2 + B1 + P2[:, :hid] + P1[:, hid:] + b
    pooled = (jnp.maximum(u, 0.0) + jnp.maximum(w, 0.0)) * 0.5
    hid_act = jnp.maximum(
        jnp.dot(pooled, W1_ref[...], preferred_element_type=jnp.float32)
        + b1_ref[...], 0.0)
    score = (jnp.dot(hid_act, w2_ref[...], preferred_element_type=jnp.float32)
             + b2_ref[...])
    out_ref[0] = score.T


def kernel(cat_id1, visual1, cat_id2, visual2, cat_table, W_vis, b_vis,
           W_self, W_nbr, b_conv, W1, b1, w2, b2):
    n, vis = visual1.shape
    ncat, emb = cat_table.shape
    hid = W1.shape[0]
    R = 1000
    nb = n // R

    n_pad = ((n + _NW * _CHUNK - 1) // (_NW * _CHUNK)) * (_NW * _CHUNK)
    b_per_w = n_pad // _NW

    c1 = jnp.pad(cat_id1.astype(jnp.int32), (0, n_pad - n))
    c2 = jnp.pad(cat_id2.astype(jnp.int32), (0, n_pad - n))

    AB, bias, TT = pl.pallas_call(
        lambda *refs: _prep_body(*refs, emb=emb, hid=hid),
        out_shape=[
            jax.ShapeDtypeStruct((vis, 2 * hid), jnp.float32),
            jax.ShapeDtypeStruct((1, hid), jnp.float32),
            jax.ShapeDtypeStruct((ncat, hid), jnp.int32),
        ],
    )(cat_table, W_vis, W_self, W_nbr,
      b_vis.reshape(1, emb), b_conv.reshape(1, hid))

    mesh = plsc.VectorSubcoreMesh(core_axis_name="c", subcore_axis_name="s")
    g1, g2 = pl.kernel(
        functools.partial(_sc_gather_body, b_per_w=b_per_w),
        mesh=mesh,
        out_type=[
            jax.ShapeDtypeStruct((n_pad, hid), jnp.int32),
            jax.ShapeDtypeStruct((n_pad, hid), jnp.int32),
        ],
        scratch_types=[
            pltpu.VMEM((2, b_per_w), jnp.int32),
            pltpu.VMEM((_NBUF, _CHUNK, hid), jnp.int32),
            pltpu.SemaphoreType.DMA,
            pltpu.SemaphoreType.DMA,
            pltpu.SemaphoreType.DMA,
            pltpu.SemaphoreType.DMA,
            pltpu.SemaphoreType.DMA,
            pltpu.SemaphoreType.DMA,
            pltpu.SemaphoreType.DMA,
            pltpu.SemaphoreType.DMA,
        ],
    )(TT, c1, c2)

    grid = (nb,)
    out = pl.pallas_call(
        lambda *refs: _main_body(*refs, hid=hid),
        grid=grid,
        in_specs=[
            pl.BlockSpec((R, vis), lambda i: (i, 0)),
            pl.BlockSpec((R, vis), lambda i: (i, 0)),
            pl.BlockSpec((R, hid), lambda i: (i, 0)),
            pl.BlockSpec((R, hid), lambda i: (i, 0)),
            pl.BlockSpec((vis, 2 * hid), lambda i: (0, 0)),
            pl.BlockSpec((1, hid), lambda i: (0, 0)),
            pl.BlockSpec((hid, hid), lambda i: (0, 0)),
            pl.BlockSpec((1, hid), lambda i: (0, 0)),
            pl.BlockSpec((hid, 1), lambda i: (0, 0)),
            pl.BlockSpec((1, 1), lambda i: (0, 0)),
        ],
        out_specs=pl.BlockSpec((1, 1, R), lambda i: (i, 0, 0)),
        out_shape=jax.ShapeDtypeStruct((nb, 1, R), jnp.float32),
    )(visual1, visual2, g1, g2, AB, bias, W1,
      b1.reshape(1, hid), w2.reshape(hid, 1), b2.reshape(1, 1))

    return out.reshape(n)


# speedup vs baseline: 2.2295x; 2.2168x over previous
import jax
import jax.numpy as jnp
from jax.experimental import pallas as pl
from jax.experimental.pallas import tpu as pltpu


def _prep_body(Wvis_ref, Wself_ref, Wnbr_ref, bvis_ref, bconv_ref,
               AB_ref, bias_ref, emb, hid):
    Wsv = Wself_ref[emb:, :]
    Wnv = Wnbr_ref[emb:, :]
    Wvis = Wvis_ref[...]
    AB_ref[:, :hid] = jnp.dot(Wvis, Wsv, preferred_element_type=jnp.float32)
    AB_ref[:, hid:] = jnp.dot(Wvis, Wnv, preferred_element_type=jnp.float32)
    bias_ref[...] = (jnp.dot(bvis_ref[...], Wsv + Wnv,
                             preferred_element_type=jnp.float32)
                     + bconv_ref[...])


def _main_body(v1_ref, v2_ref, c1_ref, c2_ref, tblT_ref, WW_ref, AB_ref,
               bias_ref, W1_ref, b1_ref, w2_ref, b2_ref, out_ref,
               ncat_pad, hid):
    r = v1_ref.shape[0]
    ids1 = c1_ref[0]
    ids2 = c2_ref[0]
    iota = jax.lax.broadcasted_iota(jnp.int32, (ncat_pad, r), 0)
    oh1T = (ids1 == iota).astype(jnp.bfloat16)
    oh2T = (ids2 == iota).astype(jnp.bfloat16)
    tblT = tblT_ref[...]
    e1 = jnp.dot(tblT, oh1T, preferred_element_type=jnp.float32).T
    e2 = jnp.dot(tblT, oh2T, preferred_element_type=jnp.float32).T
    WW = WW_ref[...]
    Q1 = jnp.dot(e1, WW, preferred_element_type=jnp.float32)
    Q2 = jnp.dot(e2, WW, preferred_element_type=jnp.float32)
    AB = AB_ref[...]
    P1 = jnp.dot(v1_ref[...], AB, preferred_element_type=jnp.float32)
    P2 = jnp.dot(v2_ref[...], AB, preferred_element_type=jnp.float32)
    b = bias_ref[...]
    u = Q1[:, :hid] + Q2[:, hid:] + P1[:, :hid] + P2[:, hid:] + b
    w = Q2[:, :hid] + Q1[:, hid:] + P2[:, :hid] + P1[:, hid:] + b
    pooled = (jnp.maximum(u, 0.0) + jnp.maximum(w, 0.0)) * 0.5
    hid_act = jnp.maximum(
        jnp.dot(pooled, W1_ref[...], preferred_element_type=jnp.float32)
        + b1_ref[...], 0.0)
    score = (jnp.dot(hid_act, w2_ref[...], preferred_element_type=jnp.float32)
             + b2_ref[...])
    out_ref[0] = score.T


def kernel(cat_id1, visual1, cat_id2, visual2, cat_table, W_vis, b_vis,
           W_self, W_nbr, b_conv, W1, b1, w2, b2):
    n, vis = visual1.shape
    ncat, emb = cat_table.shape
    hid = W1.shape[0]
    ncat_pad = ((ncat + 127) // 128) * 128
    R = 1000
    nb = n // R

    c1 = cat_id1.astype(jnp.int32).reshape(nb, 1, R)
    c2 = cat_id2.astype(jnp.int32).reshape(nb, 1, R)
    tblT = jnp.pad(cat_table, ((0, ncat_pad - ncat), (0, 0))).astype(
        jnp.bfloat16).T
    WW = jnp.concatenate([W_self[:emb], W_nbr[:emb]], axis=1)

    AB, bias = pl.pallas_call(
        lambda *refs: _prep_body(*refs, emb=emb, hid=hid),
        out_shape=[
            jax.ShapeDtypeStruct((vis, 2 * hid), jnp.float32),
            jax.ShapeDtypeStruct((1, hid), jnp.float32),
        ],
    )(W_vis, W_self, W_nbr, b_vis.reshape(1, emb), b_conv.reshape(1, hid))

    grid = (nb,)
    out = pl.pallas_call(
        lambda *refs: _main_body(*refs, ncat_pad=ncat_pad, hid=hid),
        grid=grid,
        in_specs=[
            pl.BlockSpec((R, vis), lambda i: (i, 0)),
            pl.BlockSpec((R, vis), lambda i: (i, 0)),
            pl.BlockSpec((1, 1, R), lambda i: (i, 0, 0)),
            pl.BlockSpec((1, 1, R), lambda i: (i, 0, 0)),
            pl.BlockSpec((emb, ncat_pad), lambda i: (0, 0)),
            pl.BlockSpec((emb, 2 * hid), lambda i: (0, 0)),
            pl.BlockSpec((vis, 2 * hid), lambda i: (0, 0)),
            pl.BlockSpec((1, hid), lambda i: (0, 0)),
            pl.BlockSpec((hid, hid), lambda i: (0, 0)),
            pl.BlockSpec((1, hid), lambda i: (0, 0)),
            pl.BlockSpec((hid, 1), lambda i: (0, 0)),
            pl.BlockSpec((1, 1), lambda i: (0, 0)),
        ],
        out_specs=pl.BlockSpec((1, 1, R), lambda i: (i, 0, 0)),
        out_shape=jax.ShapeDtypeStruct((nb, 1, R), jnp.float32),
    )(visual1, visual2, c1, c2, tblT, WW, AB, bias, W1,
      b1.reshape(1, hid), w2.reshape(hid, 1), b2.reshape(1, 1))

    return out.reshape(n)


# speedup vs baseline: 2.5281x; 1.1339x over previous
import jax
import jax.numpy as jnp
from jax.experimental import pallas as pl
from jax.experimental.pallas import tpu as pltpu


def _prep_body(Wvis_ref, Wself_ref, Wnbr_ref, bvis_ref, bconv_ref,
               AB_ref, bias_ref, emb, hid):
    Wsv = Wself_ref[emb:, :]
    Wnv = Wnbr_ref[emb:, :]
    Wvis = Wvis_ref[...]
    AB_ref[:, :hid] = jnp.dot(Wvis, Wsv, preferred_element_type=jnp.float32)
    AB_ref[:, hid:] = jnp.dot(Wvis, Wnv, preferred_element_type=jnp.float32)
    bias_ref[...] = (jnp.dot(bvis_ref[...], Wsv + Wnv,
                             preferred_element_type=jnp.float32)
                     + bconv_ref[...])


def _main_body(v1_ref, v2_ref, c1_ref, c2_ref, tblT_ref, WW_ref, AB_ref,
               bias_ref, W1_ref, b1_ref, w2_ref, b2_ref, out_ref,
               ncat_pad, hid):
    r = v1_ref.shape[0]
    ids1 = c1_ref[0]
    ids2 = c2_ref[0]
    iota = jax.lax.broadcasted_iota(jnp.int32, (ncat_pad, r), 0)
    oh1T = (ids1 == iota).astype(jnp.bfloat16)
    oh2T = (ids2 == iota).astype(jnp.bfloat16)
    tblT = tblT_ref[...]
    e1 = jnp.dot(tblT, oh1T, preferred_element_type=jnp.float32).T
    e2 = jnp.dot(tblT, oh2T, preferred_element_type=jnp.float32).T
    WW = WW_ref[...]
    Q1 = jnp.dot(e1, WW, preferred_element_type=jnp.float32)
    Q2 = jnp.dot(e2, WW, preferred_element_type=jnp.float32)
    AB = AB_ref[...]
    P1 = jnp.dot(v1_ref[...], AB, preferred_element_type=jnp.float32)
    P2 = jnp.dot(v2_ref[...], AB, preferred_element_type=jnp.float32)
    b = bias_ref[...]
    u = Q1[:, :hid] + Q2[:, hid:] + P1[:, :hid] + P2[:, hid:] + b
    w = Q2[:, :hid] + Q1[:, hid:] + P2[:, :hid] + P1[:, hid:] + b
    pooled = (jnp.maximum(u, 0.0) + jnp.maximum(w, 0.0)) * 0.5
    hid_act = jnp.maximum(
        jnp.dot(pooled, W1_ref[...], preferred_element_type=jnp.float32)
        + b1_ref[...], 0.0)
    score = (jnp.dot(hid_act, w2_ref[...], preferred_element_type=jnp.float32)
             + b2_ref[...])
    out_ref[0] = score.T


def kernel(cat_id1, visual1, cat_id2, visual2, cat_table, W_vis, b_vis,
           W_self, W_nbr, b_conv, W1, b1, w2, b2):
    n, vis = visual1.shape
    ncat, emb = cat_table.shape
    hid = W1.shape[0]
    ncat_pad = ((ncat + 127) // 128) * 128
    R = 2000
    nb = n // R

    c1 = cat_id1.astype(jnp.int32).reshape(nb, 1, R)
    c2 = cat_id2.astype(jnp.int32).reshape(nb, 1, R)
    tblT = jnp.pad(cat_table, ((0, ncat_pad - ncat), (0, 0))).astype(
        jnp.bfloat16).T
    WW = jnp.concatenate([W_self[:emb], W_nbr[:emb]], axis=1)

    AB, bias = pl.pallas_call(
        lambda *refs: _prep_body(*refs, emb=emb, hid=hid),
        out_shape=[
            jax.ShapeDtypeStruct((vis, 2 * hid), jnp.float32),
            jax.ShapeDtypeStruct((1, hid), jnp.float32),
        ],
    )(W_vis, W_self, W_nbr, b_vis.reshape(1, emb), b_conv.reshape(1, hid))

    grid = (nb,)
    out = pl.pallas_call(
        lambda *refs: _main_body(*refs, ncat_pad=ncat_pad, hid=hid),
        grid=grid,
        in_specs=[
            pl.BlockSpec((R, vis), lambda i: (i, 0)),
            pl.BlockSpec((R, vis), lambda i: (i, 0)),
            pl.BlockSpec((1, 1, R), lambda i: (i, 0, 0)),
            pl.BlockSpec((1, 1, R), lambda i: (i, 0, 0)),
            pl.BlockSpec((emb, ncat_pad), lambda i: (0, 0)),
            pl.BlockSpec((emb, 2 * hid), lambda i: (0, 0)),
            pl.BlockSpec((vis, 2 * hid), lambda i: (0, 0)),
            pl.BlockSpec((1, hid), lambda i: (0, 0)),
            pl.BlockSpec((hid, hid), lambda i: (0, 0)),
            pl.BlockSpec((1, hid), lambda i: (0, 0)),
            pl.BlockSpec((hid, 1), lambda i: (0, 0)),
            pl.BlockSpec((1, 1), lambda i: (0, 0)),
        ],
        out_specs=pl.BlockSpec((1, 1, R), lambda i: (i, 0, 0)),
        out_shape=jax.ShapeDtypeStruct((nb, 1, R), jnp.float32),
    )(visual1, visual2, c1, c2, tblT, WW, AB, bias, W1,
      b1.reshape(1, hid), w2.reshape(hid, 1), b2.reshape(1, 1))

    return out.reshape(n)


# speedup vs baseline: 2.6408x; 1.0446x over previous
import jax
import jax.numpy as jnp
from jax.experimental import pallas as pl
from jax.experimental.pallas import tpu as pltpu


def _prep_body(Wvis_ref, Wself_ref, Wnbr_ref, bvis_ref, bconv_ref,
               AB_ref, bias_ref, emb, hid):
    Wsv = Wself_ref[emb:, :]
    Wnv = Wnbr_ref[emb:, :]
    Wvis = Wvis_ref[...]
    AB_ref[:, :hid] = jnp.dot(Wvis, Wsv, preferred_element_type=jnp.float32)
    AB_ref[:, hid:] = jnp.dot(Wvis, Wnv, preferred_element_type=jnp.float32)
    bias_ref[...] = (jnp.dot(bvis_ref[...], Wsv + Wnv,
                             preferred_element_type=jnp.float32)
                     + bconv_ref[...])


def _main_body(v1_ref, v2_ref, c1_ref, c2_ref, tblT_ref, WW_ref, AB_ref,
               bias_ref, W1_ref, b1_ref, w2_ref, b2_ref, out_ref,
               ncat_pad, hid):
    r = v1_ref.shape[0]
    ids1 = c1_ref[0]
    ids2 = c2_ref[0]
    iota = jax.lax.broadcasted_iota(jnp.int32, (ncat_pad, r), 0)
    oh1T = (ids1 == iota).astype(jnp.bfloat16)
    oh2T = (ids2 == iota).astype(jnp.bfloat16)
    tblT = tblT_ref[...]
    e1 = jnp.dot(tblT, oh1T, preferred_element_type=jnp.float32).T
    e2 = jnp.dot(tblT, oh2T, preferred_element_type=jnp.float32).T
    WW = WW_ref[...]
    Q1 = jnp.dot(e1, WW, preferred_element_type=jnp.float32)
    Q2 = jnp.dot(e2, WW, preferred_element_type=jnp.float32)
    AB = AB_ref[...]
    P1 = jnp.dot(v1_ref[...], AB, preferred_element_type=jnp.float32)
    P2 = jnp.dot(v2_ref[...], AB, preferred_element_type=jnp.float32)
    b = bias_ref[...]
    u = Q1[:, :hid] + Q2[:, hid:] + P1[:, :hid] + P2[:, hid:] + b
    w = Q2[:, :hid] + Q1[:, hid:] + P2[:, :hid] + P1[:, hid:] + b
    pooled = (jnp.maximum(u, 0.0) + jnp.maximum(w, 0.0)) * 0.5
    hid_act = jnp.maximum(
        jnp.dot(pooled, W1_ref[...], preferred_element_type=jnp.float32)
        + b1_ref[...], 0.0)
    score = (jnp.dot(hid_act, w2_ref[...], preferred_element_type=jnp.float32)
             + b2_ref[...])
    out_ref[0] = score.T


def kernel(cat_id1, visual1, cat_id2, visual2, cat_table, W_vis, b_vis,
           W_self, W_nbr, b_conv, W1, b1, w2, b2):
    n, vis = visual1.shape
    ncat, emb = cat_table.shape
    hid = W1.shape[0]
    ncat_pad = ((ncat + 127) // 128) * 128
    R = 4000
    nb = n // R

    c1 = cat_id1.astype(jnp.int32).reshape(nb, 1, R)
    c2 = cat_id2.astype(jnp.int32).reshape(nb, 1, R)
    tblT = jnp.pad(cat_table, ((0, ncat_pad - ncat), (0, 0))).astype(
        jnp.bfloat16).T
    WW = jnp.concatenate([W_self[:emb], W_nbr[:emb]], axis=1)

    AB, bias = pl.pallas_call(
        lambda *refs: _prep_body(*refs, emb=emb, hid=hid),
        out_shape=[
            jax.ShapeDtypeStruct((vis, 2 * hid), jnp.float32),
            jax.ShapeDtypeStruct((1, hid), jnp.float32),
        ],
    )(W_vis, W_self, W_nbr, b_vis.reshape(1, emb), b_conv.reshape(1, hid))

    grid = (nb,)
    out = pl.pallas_call(
        lambda *refs: _main_body(*refs, ncat_pad=ncat_pad, hid=hid),
        grid=grid,
        in_specs=[
            pl.BlockSpec((R, vis), lambda i: (i, 0)),
            pl.BlockSpec((R, vis), lambda i: (i, 0)),
            pl.BlockSpec((1, 1, R), lambda i: (i, 0, 0)),
            pl.BlockSpec((1, 1, R), lambda i: (i, 0, 0)),
            pl.BlockSpec((emb, ncat_pad), lambda i: (0, 0)),
            pl.BlockSpec((emb, 2 * hid), lambda i: (0, 0)),
            pl.BlockSpec((vis, 2 * hid), lambda i: (0, 0)),
            pl.BlockSpec((1, hid), lambda i: (0, 0)),
            pl.BlockSpec((hid, hid), lambda i: (0, 0)),
            pl.BlockSpec((1, hid), lambda i: (0, 0)),
            pl.BlockSpec((hid, 1), lambda i: (0, 0)),
            pl.BlockSpec((1, 1), lambda i: (0, 0)),
        ],
        out_specs=pl.BlockSpec((1, 1, R), lambda i: (i, 0, 0)),
        out_shape=jax.ShapeDtypeStruct((nb, 1, R), jnp.float32),
    )(visual1, visual2, c1, c2, tblT, WW, AB, bias, W1,
      b1.reshape(1, hid), w2.reshape(hid, 1), b2.reshape(1, 1))

    return out.reshape(n)
